# baseline (speedup 1.0000x reference)
import functools

import jax
import jax.numpy as jnp
from jax.experimental import pallas as pl


BLK_B = 512


def _mm_kernel(c0_ref, c1_ref, emb_ref, out0_ref, out1_ref):
    emb = emb_ref[:].astype(jnp.bfloat16)
    out0_ref[:] = jnp.dot(c0_ref[:].astype(jnp.bfloat16), emb,
                          preferred_element_type=jnp.float32)
    out1_ref[:] = jnp.dot(c1_ref[:].astype(jnp.bfloat16), emb,
                          preferred_element_type=jnp.float32)


@functools.partial(jax.jit, static_argnames=())
def kernel(element_counts, emb):
    B, E = element_counts.shape
    D = emb.shape[1]
    half = B // 2
    steps = half // BLK_B
    out0, out1 = pl.pallas_call(
        _mm_kernel,
        grid=(steps,),
        in_specs=[
            pl.BlockSpec((BLK_B, E), lambda i: (i, 0)),
            pl.BlockSpec((BLK_B, E), lambda i, s=steps: (i + s, 0)),
            pl.BlockSpec((E, D), lambda i: (0, 0)),
        ],
        out_specs=[
            pl.BlockSpec((BLK_B, D), lambda i: (i, 0)),
            pl.BlockSpec((BLK_B, D), lambda i: (i, 0)),
        ],
        out_shape=[
            jax.ShapeDtypeStruct((half, D), jnp.float32),
            jax.ShapeDtypeStruct((half, D), jnp.float32),
        ],
    )(element_counts, element_counts, emb)
    return jnp.concatenate([out0, out1], axis=0)


# speedup vs baseline: 1.0136x; 1.0136x over previous
import functools

import jax
import jax.numpy as jnp
from jax.experimental import pallas as pl


BLK_B = 1024


def _mm_kernel(c0_ref, c1_ref, emb_ref, out0_ref, out1_ref):
    emb = emb_ref[:].astype(jnp.bfloat16)
    out0_ref[:] = jnp.dot(c0_ref[:].astype(jnp.bfloat16), emb,
                          preferred_element_type=jnp.float32)
    out1_ref[:] = jnp.dot(c1_ref[:].astype(jnp.bfloat16), emb,
                          preferred_element_type=jnp.float32)


@functools.partial(jax.jit, static_argnames=())
def kernel(element_counts, emb):
    B, E = element_counts.shape
    D = emb.shape[1]
    half = B // 2
    steps = half // BLK_B
    out0, out1 = pl.pallas_call(
        _mm_kernel,
        grid=(steps,),
        in_specs=[
            pl.BlockSpec((BLK_B, E), lambda i: (i, 0)),
            pl.BlockSpec((BLK_B, E), lambda i, s=steps: (i + s, 0)),
            pl.BlockSpec((E, D), lambda i: (0, 0)),
        ],
        out_specs=[
            pl.BlockSpec((BLK_B, D), lambda i: (i, 0)),
            pl.BlockSpec((BLK_B, D), lambda i: (i, 0)),
        ],
        out_shape=[
            jax.ShapeDtypeStruct((half, D), jnp.float32),
            jax.ShapeDtypeStruct((half, D), jnp.float32),
        ],
    )(element_counts, element_counts, emb)
    return jnp.concatenate([out0, out1], axis=0)
